# baseline (speedup 1.0000x reference)
import functools

import jax
import jax.numpy as jnp
from jax import lax
from jax.experimental import pallas as pl
from jax.experimental.pallas import tpu as pltpu
from jax.experimental.pallas import tpu_sc as plsc

NN = 10000
EE = 320000
DD = 128
CC = 40
EPS = 1e-5
ISQ = float(1.0 / (1.0 + EPS) ** 0.5)
QS = 256.0
IQS = 1.0 / QS

NC = 2
NS = 16
NW = NC * NS
NP = 10240
RPT = NP // NS
EPW = EE // NW
K = 80
NCHUNK = EPW // K

BB = 1000
GRID = NN // BB


def _sc_mesh():
    return plsc.VectorSubcoreMesh(
        core_axis_name="c", subcore_axis_name="s", num_cores=NC, num_subcores=NS
    )


def _sc_segment_pass(feat, src, dst, zeros2, zeros1, ones, with_counts):
    out_type = [jax.ShapeDtypeStruct((NC, NP, DD), jnp.float32)]
    scratch = [
        pltpu.VMEM((K,), jnp.int32),
        pltpu.VMEM((K,), jnp.int32),
        pltpu.VMEM((K,), jnp.int32),
        pltpu.VMEM((K,), jnp.int32),
        pltpu.VMEM((K, DD), jnp.float32),
        pltpu.VMEM((K, DD), jnp.float32),
        pltpu.VMEM_SHARED((NP, DD), jnp.float32),
        pltpu.SemaphoreType.DMA,
        pltpu.SemaphoreType.DMA,
        pltpu.SemaphoreType.DMA,
        pltpu.SemaphoreType.DMA,
        pltpu.SemaphoreType.DMA,
        pltpu.SemaphoreType.DMA,
        pltpu.SemaphoreType.DMA,
        pltpu.SemaphoreType.DMA,
        pltpu.SemaphoreType.DMA,
        pltpu.SemaphoreType.DMA,
        pltpu.SemaphoreType.DMA,
        pltpu.SemaphoreType.DMA,
    ]
    if with_counts:
        out_type.append(jax.ShapeDtypeStruct((NC * NP,), jnp.float32))
        scratch += [
            pltpu.VMEM((K,), jnp.float32),
            pltpu.VMEM_SHARED((NP,), jnp.float32),
        ]

    def body(*refs):
        if with_counts:
            (feat_h, src_h, dst_h, z2_h, z1_h, ones_h, sums_h, cnts_h,
             s_a, s_b, d_a, d_b, rows_a, rows_b, acc,
             sem_ra, sem_rb, sem_sa, sem_sb, sem_da, sem_db,
             sem_wa, sem_wb, sem_ca, sem_cb, sem_r2a, sem_r2b,
             ones_v, cacc) = refs
        else:
            (feat_h, src_h, dst_h, z2_h, sums_h,
             s_a, s_b, d_a, d_b, rows_a, rows_b, acc,
             sem_ra, sem_rb, sem_sa, sem_sb, sem_da, sem_db,
             sem_wa, sem_wb, sem_ca, sem_cb, sem_r2a, sem_r2b) = refs
        sv = [s_a, s_b]
        dv = [d_a, d_b]
        rv = [rows_a, rows_b]
        sem_r = [sem_ra, sem_rb]
        sem_r2 = [sem_r2a, sem_r2b]
        sem_s = [sem_sa, sem_sb]
        sem_d = [sem_da, sem_db]
        sem_w = [sem_wa, sem_wb]
        sem_c = [sem_ca, sem_cb]
        c = lax.axis_index("c")
        s = lax.axis_index("s")
        wid = c * NS + s
        rbase = pl.multiple_of(s * RPT, 8)
        pltpu.sync_copy(z2_h, acc.at[pl.ds(rbase, RPT)])
        if with_counts:
            pltpu.sync_copy(z1_h, cacc.at[pl.ds(rbase, RPT)])
            pltpu.sync_copy(ones_h, ones_v)
        plsc.subcore_barrier()

        ebase = wid * EPW

        def ioff(j):
            return pl.multiple_of(ebase + j * K, 8)

        def clamp(j):
            return jnp.minimum(j, NCHUNK - 1)

        def iload(h, j, buf, sem):
            pltpu.async_copy(h.at[pl.ds(ioff(j), K)], buf, sem)

        def iwait(h, buf, sem):
            pltpu.make_async_copy(h.at[pl.ds(ioff(0), K)], buf, sem).wait()

        KH = K // 2

        def gath(x):
            pltpu.make_async_copy(feat_h.at[sv[x].at[pl.ds(0, KH)]],
                                  rv[x].at[pl.ds(0, KH)], sem_r[x]).start()
            pltpu.make_async_copy(feat_h.at[sv[x].at[pl.ds(KH, KH)]],
                                  rv[x].at[pl.ds(KH, KH)], sem_r2[x]).start()

        def gwait(x):
            pltpu.make_async_copy(feat_h.at[sv[x].at[pl.ds(0, KH)]],
                                  rv[x].at[pl.ds(0, KH)], sem_r[x]).wait()
            pltpu.make_async_copy(feat_h.at[sv[x].at[pl.ds(KH, KH)]],
                                  rv[x].at[pl.ds(KH, KH)], sem_r2[x]).wait()

        def scat_start(x):
            pltpu.make_async_copy(rv[x], acc.at[dv[x]],
                                  sem_w[x]).start(add=True)
            if with_counts:
                pltpu.make_async_copy(ones_v, cacc.at[dv[x]],
                                      sem_c[x]).start(add=True)

        def scat_wait(x):
            pltpu.make_async_copy(rv[x], acc.at[dv[x]], sem_w[x]).wait()
            if with_counts:
                pltpu.make_async_copy(ones_v, cacc.at[dv[x]],
                                      sem_c[x]).wait()

        def half(j, x, first):
            y = 1 - x
            if not first:
                scat_wait(y)
            iwait(src_h, sv[y], sem_s[y])
            gath(y)
            gwait(x)
            iwait(dst_h, dv[x], sem_d[x])
            scat_start(x)
            iload(src_h, clamp(j + 2), sv[x], sem_s[x])
            iload(dst_h, clamp(j + 1), dv[y], sem_d[y])

        pltpu.sync_copy(src_h.at[pl.ds(ioff(0), K)], s_a)
        gath(0)
        iload(dst_h, 0, d_a, sem_da)
        iload(src_h, 1, s_b, sem_sb)
        half(0, 0, True)

        def step(i, carry):
            half(2 * i + 1, 1, False)
            half(2 * i + 2, 0, False)
            return carry

        lax.fori_loop(0, (NCHUNK - 1) // 2, step, 0)
        scat_wait(0)
        gwait(1)
        iwait(src_h, s_a, sem_sa)
        iwait(dst_h, d_b, sem_db)
        plsc.subcore_barrier()
        pltpu.sync_copy(acc.at[pl.ds(rbase, RPT)],
                        sums_h.at[c, pl.ds(rbase, RPT)])
        if with_counts:
            cb = pl.multiple_of(c * NP + rbase, 8)
            pltpu.sync_copy(cacc.at[pl.ds(rbase, RPT)],
                            cnts_h.at[pl.ds(cb, RPT)])

    fn = pl.kernel(body, out_type=tuple(out_type), mesh=_sc_mesh(),
                   scratch_types=scratch)
    if with_counts:
        return fn(feat, src, dst, zeros2, zeros1, ones)
    return fn(feat, src, dst, zeros2)[0]


def _dense1_body(x, pa, cn, wl, wr, wres, g, b, br, c1_o, h_o):
    s1 = pa[0] + pa[1]
    cnt = cn[0] + cn[1]
    rcp = 1.0 / jnp.maximum(cnt, 1.0)
    agg = s1 * rcp
    t = (jnp.dot(agg, wl[...], preferred_element_type=jnp.float32)
         + jnp.dot(x[...], wr[...], preferred_element_type=jnp.float32))
    t = g[...] * (t * ISQ) + b[...]
    c1 = jnp.maximum(t, 0.0)
    c1_o[...] = c1
    h = c1 + jnp.dot(x[...], wres[...],
                     preferred_element_type=jnp.float32) + br[...]
    h_o[...] = h


def _dense2_body(x, c1, h, pb, cn, wl2, wr2, g2, b2,
                 w0x, w0c1, w0c2, b0, gm, bm, w1, b1m, out_o):
    s2 = pb[0] + pb[1]
    cnt = cn[0] + cn[1]
    rcp = 1.0 / jnp.maximum(cnt, 1.0)
    agg2 = s2 * rcp
    t = (jnp.dot(agg2, wl2[...], preferred_element_type=jnp.float32)
         + jnp.dot(h[...], wr2[...], preferred_element_type=jnp.float32))
    t = g2[...] * (t * ISQ) + b2[...]
    c2 = jnp.maximum(t, 0.0)
    zz = (jnp.dot(x[...], w0x[...], preferred_element_type=jnp.float32)
          + jnp.dot(c1[...], w0c1[...], preferred_element_type=jnp.float32)
          + jnp.dot(c2[...], w0c2[...], preferred_element_type=jnp.float32)
          + b0[...])
    z1 = jnp.maximum(gm[...] * (zz * ISQ) + bm[...], 0.0)
    full = jnp.dot(z1, w1[...], preferred_element_type=jnp.float32) + b1m[...]
    out_o[...] = full[:, :CC]


def _row_spec(k=DD):
    return pl.BlockSpec((BB, k), lambda i: (i, 0))


def _w_spec():
    return pl.BlockSpec((DD, DD), lambda i: (0, 0))


def _v_spec(k=DD):
    return pl.BlockSpec((1, k), lambda i: (0, 0))


def _pad_cols(a, k=DD):
    return jnp.pad(a, [(0, 0)] * (a.ndim - 1) + [(0, k - a.shape[-1])])


def kernel(x, edge_index, Wl1, Wr1, g1, b1, Wl2, Wr2, g2, b2,
           Wres, bres, Wm0, bm0, gm, bm, Wm1, bm1):
    f32 = jnp.float32
    src = edge_index[0]
    dst = edge_index[1]
    zeros2 = jnp.zeros((RPT, DD), f32)
    zeros1 = jnp.zeros((RPT,), f32)
    ones = jnp.ones((K,), f32)
    sums_a, cnts = _sc_segment_pass(x, src, dst, zeros2, zeros1, ones, True)
    pa = sums_a
    cn = cnts.reshape(NC, NP, 1)

    cn_spec = pl.BlockSpec((NC, BB, 1), lambda i: (0, i, 0))
    pa_spec = pl.BlockSpec((NC, BB, DD), lambda i: (0, i, 0))
    c1, h = pl.pallas_call(
        _dense1_body,
        grid=(GRID,),
        in_specs=[_row_spec(), pa_spec, cn_spec, _w_spec(), _w_spec(),
                  _w_spec(), _v_spec(), _v_spec(), _v_spec()],
        out_specs=(_row_spec(), _row_spec()),
        out_shape=(jax.ShapeDtypeStruct((NN, DD), f32),
                   jax.ShapeDtypeStruct((NN, DD), f32)),
    )(x, pa, cn, Wl1, Wr1, Wres, g1.reshape(1, DD), b1.reshape(1, DD),
      bres.reshape(1, DD))

    pb = _sc_segment_pass(h, src, dst, zeros2, None, None, False)

    w0x = _pad_cols(Wm0[0:DD])
    w0c1 = _pad_cols(Wm0[DD:2 * DD])
    w0c2 = _pad_cols(Wm0[2 * DD:3 * DD])
    b0 = _pad_cols(bm0.reshape(1, -1))
    gmp = _pad_cols(gm.reshape(1, -1))
    bmp = _pad_cols(bm.reshape(1, -1))
    w1 = jnp.pad(Wm1, [(0, DD - Wm1.shape[0]), (0, DD - Wm1.shape[1])])
    b1m = _pad_cols(bm1.reshape(1, -1))
    out = pl.pallas_call(
        _dense2_body,
        grid=(GRID,),
        in_specs=[_row_spec(), _row_spec(), _row_spec(), pa_spec, cn_spec,
                  _w_spec(), _w_spec(), _v_spec(), _v_spec(),
                  _w_spec(), _w_spec(), _w_spec(), _v_spec(), _v_spec(),
                  _v_spec(), _w_spec(), _v_spec()],
        out_specs=_row_spec(CC),
        out_shape=jax.ShapeDtypeStruct((NN, CC), f32),
    )(x, c1, h, pb, cn, Wl2, Wr2, g2.reshape(1, DD), b2.reshape(1, DD),
      w0x, w0c1, w0c2, b0, gmp, bmp, w1, b1m)
    return out


# speedup vs baseline: 1.0084x; 1.0084x over previous
import functools

import jax
import jax.numpy as jnp
from jax import lax
from jax.experimental import pallas as pl
from jax.experimental.pallas import tpu as pltpu
from jax.experimental.pallas import tpu_sc as plsc

NN = 10000
EE = 320000
DD = 128
CC = 40
EPS = 1e-5
ISQ = float(1.0 / (1.0 + EPS) ** 0.5)
QS = 256.0
IQS = 1.0 / QS

NC = 2
NS = 16
NW = NC * NS
NP = 10240
RPT = NP // NS
EPW = EE // NW
K = 80
NCHUNK = EPW // K

BB = 1000
GRID = NN // BB


def _sc_mesh():
    return plsc.VectorSubcoreMesh(
        core_axis_name="c", subcore_axis_name="s", num_cores=NC, num_subcores=NS
    )


def _sc_segment_pass(feat, src, dst, zeros2, zeros1, ones, with_counts):
    out_type = [jax.ShapeDtypeStruct((NC, NP, DD), jnp.float32)]
    scratch = [
        pltpu.VMEM((K,), jnp.int32),
        pltpu.VMEM((K,), jnp.int32),
        pltpu.VMEM((K,), jnp.int32),
        pltpu.VMEM((K,), jnp.int32),
        pltpu.VMEM((K, DD), jnp.float32),
        pltpu.VMEM((K, DD), jnp.float32),
        pltpu.VMEM_SHARED((NP, DD), jnp.float32),
        pltpu.SemaphoreType.DMA,
        pltpu.SemaphoreType.DMA,
        pltpu.SemaphoreType.DMA,
        pltpu.SemaphoreType.DMA,
        pltpu.SemaphoreType.DMA,
        pltpu.SemaphoreType.DMA,
        pltpu.SemaphoreType.DMA,
        pltpu.SemaphoreType.DMA,
        pltpu.SemaphoreType.DMA,
        pltpu.SemaphoreType.DMA,
        pltpu.SemaphoreType.DMA,
        pltpu.SemaphoreType.DMA,
    ]
    if with_counts:
        out_type.append(jax.ShapeDtypeStruct((NC * NP,), jnp.float32))
        scratch += [
            pltpu.VMEM((K,), jnp.float32),
            pltpu.VMEM_SHARED((NP,), jnp.float32),
        ]

    def body(*refs):
        if with_counts:
            (feat_h, src_h, dst_h, z2_h, z1_h, ones_h, sums_h, cnts_h,
             s_a, s_b, d_a, d_b, rows_a, rows_b, acc,
             sem_ra, sem_rb, sem_sa, sem_sb, sem_da, sem_db,
             sem_wa, sem_wb, sem_ca, sem_cb, sem_r2a, sem_r2b,
             ones_v, cacc) = refs
        else:
            (feat_h, src_h, dst_h, z2_h, sums_h,
             s_a, s_b, d_a, d_b, rows_a, rows_b, acc,
             sem_ra, sem_rb, sem_sa, sem_sb, sem_da, sem_db,
             sem_wa, sem_wb, sem_ca, sem_cb, sem_r2a, sem_r2b) = refs
        sv = [s_a, s_b]
        dv = [d_a, d_b]
        rv = [rows_a, rows_b]
        sem_r = [sem_ra, sem_rb]
        sem_r2 = [sem_r2a, sem_r2b]
        sem_s = [sem_sa, sem_sb]
        sem_d = [sem_da, sem_db]
        sem_w = [sem_wa, sem_wb]
        sem_c = [sem_ca, sem_cb]
        c = lax.axis_index("c")
        s = lax.axis_index("s")
        wid = c * NS + s
        rbase = pl.multiple_of(s * RPT, 8)
        pltpu.sync_copy(z2_h, acc.at[pl.ds(rbase, RPT)])
        if with_counts:
            pltpu.sync_copy(z1_h, cacc.at[pl.ds(rbase, RPT)])
            pltpu.sync_copy(ones_h, ones_v)
        plsc.subcore_barrier()

        ebase = wid * EPW

        def ioff(j):
            return pl.multiple_of(ebase + j * K, 8)

        def clamp(j):
            return jnp.minimum(j, NCHUNK - 1)

        def iload(h, j, buf, sem):
            pltpu.async_copy(h.at[pl.ds(ioff(j), K)], buf, sem)

        def iwait(h, buf, sem):
            pltpu.make_async_copy(h.at[pl.ds(ioff(0), K)], buf, sem).wait()

        def gath(x):
            pltpu.make_async_copy(feat_h.at[sv[x]], rv[x], sem_r[x]).start()

        def gwait(x):
            pltpu.make_async_copy(feat_h.at[sv[x]], rv[x], sem_r[x]).wait()

        def scat_start(x):
            pltpu.make_async_copy(rv[x], acc.at[dv[x]],
                                  sem_w[x]).start(add=True)
            if with_counts:
                pltpu.make_async_copy(ones_v, cacc.at[dv[x]],
                                      sem_c[x]).start(add=True)

        def scat_wait(x):
            pltpu.make_async_copy(rv[x], acc.at[dv[x]], sem_w[x]).wait()
            if with_counts:
                pltpu.make_async_copy(ones_v, cacc.at[dv[x]],
                                      sem_c[x]).wait()

        def half(j, x, first):
            y = 1 - x
            if not first:
                scat_wait(y)
            iwait(src_h, sv[y], sem_s[y])
            gath(y)
            gwait(x)
            iwait(dst_h, dv[x], sem_d[x])
            scat_start(x)
            iload(src_h, clamp(j + 2), sv[x], sem_s[x])
            iload(dst_h, clamp(j + 1), dv[y], sem_d[y])

        pltpu.sync_copy(src_h.at[pl.ds(ioff(0), K)], s_a)
        gath(0)
        iload(dst_h, 0, d_a, sem_da)
        iload(src_h, 1, s_b, sem_sb)
        half(0, 0, True)

        def step(i, carry):
            half(2 * i + 1, 1, False)
            half(2 * i + 2, 0, False)
            return carry

        lax.fori_loop(0, (NCHUNK - 1) // 2, step, 0)
        scat_wait(0)
        gwait(1)
        iwait(src_h, s_a, sem_sa)
        iwait(dst_h, d_b, sem_db)
        plsc.subcore_barrier()
        pltpu.sync_copy(acc.at[pl.ds(rbase, RPT)],
                        sums_h.at[c, pl.ds(rbase, RPT)])
        if with_counts:
            cb = pl.multiple_of(c * NP + rbase, 8)
            pltpu.sync_copy(cacc.at[pl.ds(rbase, RPT)],
                            cnts_h.at[pl.ds(cb, RPT)])

    fn = pl.kernel(body, out_type=tuple(out_type), mesh=_sc_mesh(),
                   scratch_types=scratch)
    if with_counts:
        return fn(feat, src, dst, zeros2, zeros1, ones)
    return fn(feat, src, dst, zeros2)[0]


def _dense1_body(x, pa, cn, wl, wr, wres, g, b, br, c1_o, h_o):
    s1 = pa[0] + pa[1]
    cnt = cn[0] + cn[1]
    rcp = 1.0 / jnp.maximum(cnt, 1.0)
    agg = s1 * rcp
    t = (jnp.dot(agg, wl[...], preferred_element_type=jnp.float32)
         + jnp.dot(x[...], wr[...], preferred_element_type=jnp.float32))
    t = g[...] * (t * ISQ) + b[...]
    c1 = jnp.maximum(t, 0.0)
    c1_o[...] = c1
    h = c1 + jnp.dot(x[...], wres[...],
                     preferred_element_type=jnp.float32) + br[...]
    h_o[...] = h


def _dense2_body(x, c1, h, pb, cn, wl2, wr2, g2, b2,
                 w0x, w0c1, w0c2, b0, gm, bm, w1, b1m, out_o):
    s2 = pb[0] + pb[1]
    cnt = cn[0] + cn[1]
    rcp = 1.0 / jnp.maximum(cnt, 1.0)
    agg2 = s2 * rcp
    t = (jnp.dot(agg2, wl2[...], preferred_element_type=jnp.float32)
         + jnp.dot(h[...], wr2[...], preferred_element_type=jnp.float32))
    t = g2[...] * (t * ISQ) + b2[...]
    c2 = jnp.maximum(t, 0.0)
    zz = (jnp.dot(x[...], w0x[...], preferred_element_type=jnp.float32)
          + jnp.dot(c1[...], w0c1[...], preferred_element_type=jnp.float32)
          + jnp.dot(c2[...], w0c2[...], preferred_element_type=jnp.float32)
          + b0[...])
    z1 = jnp.maximum(gm[...] * (zz * ISQ) + bm[...], 0.0)
    full = jnp.dot(z1, w1[...], preferred_element_type=jnp.float32) + b1m[...]
    out_o[...] = full[:, :CC]


def _row_spec(k=DD):
    return pl.BlockSpec((BB, k), lambda i: (i, 0))


def _w_spec():
    return pl.BlockSpec((DD, DD), lambda i: (0, 0))


def _v_spec(k=DD):
    return pl.BlockSpec((1, k), lambda i: (0, 0))


def _pad_cols(a, k=DD):
    return jnp.pad(a, [(0, 0)] * (a.ndim - 1) + [(0, k - a.shape[-1])])


def kernel(x, edge_index, Wl1, Wr1, g1, b1, Wl2, Wr2, g2, b2,
           Wres, bres, Wm0, bm0, gm, bm, Wm1, bm1):
    f32 = jnp.float32
    src = edge_index[0]
    dst = edge_index[1]
    zeros2 = jnp.zeros((RPT, DD), f32)
    zeros1 = jnp.zeros((RPT,), f32)
    ones = jnp.ones((K,), f32)
    sums_a, cnts = _sc_segment_pass(x, src, dst, zeros2, zeros1, ones, True)
    pa = sums_a
    cn = cnts.reshape(NC, NP, 1)

    cn_spec = pl.BlockSpec((NC, BB, 1), lambda i: (0, i, 0))
    pa_spec = pl.BlockSpec((NC, BB, DD), lambda i: (0, i, 0))
    c1, h = pl.pallas_call(
        _dense1_body,
        grid=(GRID,),
        in_specs=[_row_spec(), pa_spec, cn_spec, _w_spec(), _w_spec(),
                  _w_spec(), _v_spec(), _v_spec(), _v_spec()],
        out_specs=(_row_spec(), _row_spec()),
        out_shape=(jax.ShapeDtypeStruct((NN, DD), f32),
                   jax.ShapeDtypeStruct((NN, DD), f32)),
    )(x, pa, cn, Wl1, Wr1, Wres, g1.reshape(1, DD), b1.reshape(1, DD),
      bres.reshape(1, DD))

    pb = _sc_segment_pass(h, src, dst, zeros2, None, None, False)

    w0x = _pad_cols(Wm0[0:DD])
    w0c1 = _pad_cols(Wm0[DD:2 * DD])
    w0c2 = _pad_cols(Wm0[2 * DD:3 * DD])
    b0 = _pad_cols(bm0.reshape(1, -1))
    gmp = _pad_cols(gm.reshape(1, -1))
    bmp = _pad_cols(bm.reshape(1, -1))
    w1 = jnp.pad(Wm1, [(0, DD - Wm1.shape[0]), (0, DD - Wm1.shape[1])])
    b1m = _pad_cols(bm1.reshape(1, -1))
    out = pl.pallas_call(
        _dense2_body,
        grid=(GRID,),
        in_specs=[_row_spec(), _row_spec(), _row_spec(), pa_spec, cn_spec,
                  _w_spec(), _w_spec(), _v_spec(), _v_spec(),
                  _w_spec(), _w_spec(), _w_spec(), _v_spec(), _v_spec(),
                  _v_spec(), _w_spec(), _v_spec()],
        out_specs=_row_spec(CC),
        out_shape=jax.ShapeDtypeStruct((NN, CC), f32),
    )(x, c1, h, pb, cn, Wl2, Wr2, g2.reshape(1, DD), b2.reshape(1, DD),
      w0x, w0c1, w0c2, b0, gmp, bmp, w1, b1m)
    return out
